# baseline (speedup 1.0000x reference)
import functools

import jax
import jax.numpy as jnp
from jax import lax
from jax.experimental import pallas as pl
from jax.experimental.pallas import tpu as pltpu
from jax.experimental.pallas import tpu_sc as plsc

NC = 2
NS = 16
CK = 128
F = 128

_MESH = plsc.VectorSubcoreMesh(core_axis_name="c", subcore_axis_name="s")


def _make_deg_kernel(ch, nr, rpt):
    hch = ch // 2

    @functools.partial(
        pl.kernel,
        out_type=jax.ShapeDtypeStruct((NC, nr, F), jnp.float32),
        mesh=_MESH,
        scratch_types=[
            pltpu.VMEM((CK,), jnp.int32),
            pltpu.VMEM((CK,), jnp.int32),
            pltpu.VMEM((CK, F), jnp.float32),
            pltpu.VMEM_SHARED((nr, F), jnp.float32),
            pltpu.SemaphoreType.DMA,
            pltpu.SemaphoreType.DMA,
        ],
    )
    def deg_kernel(dst_hbm, zeros_hbm, ones_hbm, out, dst_v, dst_w, ones_v,
                   acc, sem_a, sem_b):
        c = lax.axis_index("c")
        s = lax.axis_index("s")

        pltpu.sync_copy(zeros_hbm, acc.at[pl.ds(s * rpt, rpt)])
        pltpu.sync_copy(ones_hbm, ones_v)
        plsc.subcore_barrier()

        base0 = (c * NS + s) * hch * CK
        pltpu.async_copy(dst_hbm.at[pl.ds(base0, CK)], dst_v, sem_a)
        pltpu.async_copy(dst_hbm.at[pl.ds(base0 + CK, CK)], dst_w, sem_b)

        def body(i, carry):
            j0 = 2 * i
            pltpu.make_async_copy(dst_hbm.at[pl.ds(base0, CK)],
                                  dst_v, sem_a).wait()
            pltpu.sync_copy(ones_v, acc.at[dst_v], add=True)

            @pl.when(i + 1 < hch // 2)
            def _():
                pltpu.async_copy(dst_hbm.at[pl.ds(base0 + (j0 + 2) * CK, CK)],
                                 dst_v, sem_a)

            pltpu.make_async_copy(dst_hbm.at[pl.ds(base0, CK)],
                                  dst_w, sem_b).wait()
            pltpu.sync_copy(ones_v, acc.at[dst_w], add=True)

            @pl.when(i + 1 < hch // 2)
            def _():
                pltpu.async_copy(dst_hbm.at[pl.ds(base0 + (j0 + 3) * CK, CK)],
                                 dst_w, sem_b)
            return carry

        lax.fori_loop(0, hch // 2, body, 0)
        plsc.subcore_barrier()
        pltpu.sync_copy(acc.at[pl.ds(s * rpt, rpt)],
                        out.at[c, pl.ds(s * rpt, rpt)])

    return deg_kernel


def _make_prop_kernel(ch, nr, rpt):
    nhalf = ch // 2

    @functools.partial(
        pl.kernel,
        out_type=jax.ShapeDtypeStruct((NC, nr, F), jnp.float32),
        mesh=_MESH,
        scratch_types=[
            pltpu.VMEM((CK,), jnp.int32),
            pltpu.VMEM((CK,), jnp.int32),
            pltpu.VMEM((CK,), jnp.int32),
            pltpu.VMEM((CK,), jnp.int32),
            pltpu.VMEM((CK, F), jnp.float32),
            pltpu.VMEM((CK, F), jnp.float32),
            pltpu.VMEM_SHARED((nr, F), jnp.float32),
            pltpu.SemaphoreType.DMA,
            pltpu.SemaphoreType.DMA,
            pltpu.SemaphoreType.DMA,
            pltpu.SemaphoreType.DMA,
        ],
    )
    def prop_kernel(src_hbm, dst_hbm, y_a, y_b, zeros_hbm, out,
                    src_a, dst_a, src_b, dst_b, rows_a, rows_b, acc,
                    sem_ia, sem_ib, sem_ga, sem_gb):
        c = lax.axis_index("c")
        s = lax.axis_index("s")

        pltpu.sync_copy(zeros_hbm, acc.at[pl.ds(s * rpt, rpt)])
        plsc.subcore_barrier()

        def run(y_hbm):
            pltpu.async_copy(src_hbm.at[s, 0], src_a, sem_ia)
            pltpu.async_copy(dst_hbm.at[s, 0], dst_a, sem_ia)
            pltpu.async_copy(src_hbm.at[s, 1], src_b, sem_ib)
            pltpu.async_copy(dst_hbm.at[s, 1], dst_b, sem_ib)

            def body(i, carry):
                j0 = 2 * i
                pltpu.make_async_copy(src_hbm.at[s, j0], src_a, sem_ia).wait()
                pltpu.make_async_copy(dst_hbm.at[s, j0], dst_a, sem_ia).wait()
                pltpu.async_copy(y_hbm.at[src_a], rows_a, sem_ga)
                pltpu.make_async_copy(src_hbm.at[s, j0], src_b, sem_ib).wait()
                pltpu.make_async_copy(dst_hbm.at[s, j0], dst_b, sem_ib).wait()
                pltpu.async_copy(y_hbm.at[src_b], rows_b, sem_gb)
                pltpu.make_async_copy(y_hbm.at[src_a], rows_a, sem_ga).wait()
                pltpu.sync_copy(rows_a, acc.at[dst_a], add=True)

                @pl.when(i + 1 < nhalf)
                def _():
                    pltpu.async_copy(src_hbm.at[s, j0 + 2], src_a, sem_ia)
                    pltpu.async_copy(dst_hbm.at[s, j0 + 2], dst_a, sem_ia)

                pltpu.make_async_copy(y_hbm.at[src_b], rows_b, sem_gb).wait()
                pltpu.sync_copy(rows_b, acc.at[dst_b], add=True)

                @pl.when(i + 1 < nhalf)
                def _():
                    pltpu.async_copy(src_hbm.at[s, j0 + 3], src_b, sem_ib)
                    pltpu.async_copy(dst_hbm.at[s, j0 + 3], dst_b, sem_ib)
                return carry

            lax.fori_loop(0, nhalf, body, 0)

        @pl.when(c == 0)
        def _():
            run(y_a)

        @pl.when(c == 1)
        def _():
            run(y_b)

        plsc.subcore_barrier()
        pltpu.sync_copy(acc.at[pl.ds(s * rpt, rpt)],
                        out.at[c, pl.ds(s * rpt, rpt)])

    return prop_kernel


def _mm1(x, w1, deg_a, deg_b):
    n, k = x.shape
    bm = 1000
    grid = n // bm

    def body(x_ref, w_ref, da_ref, db_ref, oa_ref, ob_ref, od_ref):
        xw = jnp.dot(x_ref[...], w_ref[...], preferred_element_type=jnp.float32)
        dinv = lax.rsqrt(da_ref[...] + db_ref[...] + 1.0)
        od_ref[...] = dinv
        y = xw * dinv
        oa_ref[...] = y[:, :F]
        ob_ref[...] = y[:, F:]

    return pl.pallas_call(
        body,
        grid=(grid,),
        in_specs=[
            pl.BlockSpec((bm, k), lambda i: (i, 0)),
            pl.BlockSpec((k, 2 * F), lambda i: (0, 0)),
            pl.BlockSpec((bm, 1), lambda i: (i, 0)),
            pl.BlockSpec((bm, 1), lambda i: (i, 0)),
        ],
        out_specs=[pl.BlockSpec((bm, F), lambda i: (i, 0))] * 2
        + [pl.BlockSpec((bm, 1), lambda i: (i, 0))],
        out_shape=(jax.ShapeDtypeStruct((n, F), jnp.float32),) * 2
        + (jax.ShapeDtypeStruct((n, 1), jnp.float32),),
    )(x, w1, deg_a, deg_b)


def _mm2(s_a, s_b, y_a, y_b, dinv_in, b1, wmu, wlv):
    n = s_a.shape[0]
    bm = 1000
    grid = n // bm

    def body(sa_ref, sb_ref, ya_ref, yb_ref, dinv_ref, b1_ref, wmu_ref,
             wlv_ref, oa_ref, ob_ref):
        dinv = dinv_ref[...]
        b = b1_ref[...]
        ha = jnp.maximum(dinv * (sa_ref[...] + ya_ref[...]) + b[:, :F], 0.0)
        hb = jnp.maximum(dinv * (sb_ref[...] + yb_ref[...]) + b[:, F:], 0.0)
        h = jnp.concatenate([ha, hb], axis=1)
        oa_ref[...] = dinv * jnp.dot(h, wmu_ref[...],
                                     preferred_element_type=jnp.float32)
        ob_ref[...] = dinv * jnp.dot(h, wlv_ref[...],
                                     preferred_element_type=jnp.float32)

    row = lambda i: (i, 0)
    fixed = lambda i: (0, 0)
    return pl.pallas_call(
        body,
        grid=(grid,),
        in_specs=[
            pl.BlockSpec((bm, F), row),
            pl.BlockSpec((bm, F), row),
            pl.BlockSpec((bm, F), row),
            pl.BlockSpec((bm, F), row),
            pl.BlockSpec((bm, 1), row),
            pl.BlockSpec((1, 2 * F), fixed),
            pl.BlockSpec((2 * F, F), fixed),
            pl.BlockSpec((2 * F, F), fixed),
        ],
        out_specs=[pl.BlockSpec((bm, F), row)] * 2,
        out_shape=(jax.ShapeDtypeStruct((n, F), jnp.float32),) * 2,
    )(s_a, s_b, y_a, y_b, dinv_in, b1, wmu, wlv)


def _fin(s_a, s_b, y_a, y_b, dinv_in, bmu, blv):
    n = s_a.shape[0]
    bm = 1000
    grid = n // bm

    def body(sa_ref, sb_ref, ya_ref, yb_ref, dinv_ref, bmu_ref, blv_ref,
             omu_ref, olv_ref):
        dinv = dinv_ref[...]
        omu_ref[...] = dinv * (sa_ref[...] + ya_ref[...]) + bmu_ref[...]
        olv_ref[...] = dinv * (sb_ref[...] + yb_ref[...]) + blv_ref[...]

    row = lambda i: (i, 0)
    fixed = lambda i: (0, 0)
    return pl.pallas_call(
        body,
        grid=(grid,),
        in_specs=[
            pl.BlockSpec((bm, F), row),
            pl.BlockSpec((bm, F), row),
            pl.BlockSpec((bm, F), row),
            pl.BlockSpec((bm, F), row),
            pl.BlockSpec((bm, 1), row),
            pl.BlockSpec((1, F), fixed),
            pl.BlockSpec((1, F), fixed),
        ],
        out_specs=[pl.BlockSpec((bm, F), row)] * 2,
        out_shape=(jax.ShapeDtypeStruct((n, F), jnp.float32),) * 2,
    )(s_a, s_b, y_a, y_b, dinv_in, bmu, blv)


def kernel(x, edge_index, W1, b1, Wmu, bmu, Wlv, blv):
    n = x.shape[0]
    e0 = edge_index.shape[1]

    ch = -(-e0 // (NS * CK))
    ch = (ch + 3) // 4 * 4
    ep = NS * ch * CK
    rpt = ((n + NS - 1) // NS + 15) // 16 * 16
    nr = NS * rpt
    trash = n

    src = edge_index[0].astype(jnp.int32)
    dst = edge_index[1].astype(jnp.int32)
    pad = ep - e0
    src_p = jnp.concatenate([src, jnp.zeros((pad,), jnp.int32)])
    trash_rows = trash + jnp.arange(pad, dtype=jnp.int32) % (nr - n)
    dst_p = jnp.concatenate([dst, trash_rows])
    zeros128 = jnp.zeros((rpt, F), jnp.float32)
    ones128 = jnp.ones((CK, F), jnp.float32)

    deg_kernel = _make_deg_kernel(ch, nr, rpt)
    prop_kernel = _make_prop_kernel(ch, nr, rpt)

    deg_p = deg_kernel(dst_p, zeros128, ones128)
    y1a, y1b, dinv = _mm1(x, W1, deg_p[0, :n, :1], deg_p[1, :n, :1])
    src3 = src_p.reshape(NS, ch, CK)
    dst3 = dst_p.reshape(NS, ch, CK)
    s1 = prop_kernel(src3, dst3, y1a, y1b, zeros128)
    y2a, y2b = _mm2(s1[0, :n], s1[1, :n], y1a, y1b, dinv,
                    b1.reshape(1, 2 * F), Wmu, Wlv)
    s2 = prop_kernel(src3, dst3, y2a, y2b, zeros128)
    mu, logvar = _fin(s2[0, :n], s2[1, :n], y2a, y2b, dinv,
                      bmu.reshape(1, F), blv.reshape(1, F))
    return (mu, logvar)


# speedup vs baseline: 2.3111x; 2.3111x over previous
import functools

import jax
import jax.numpy as jnp
from jax import lax
from jax.experimental import pallas as pl
from jax.experimental.pallas import tpu as pltpu
from jax.experimental.pallas import tpu_sc as plsc

NC = 2
NS = 16
CK = 128
F = 128

_MESH = plsc.VectorSubcoreMesh(core_axis_name="c", subcore_axis_name="s")


def _make_deg_kernel(ch, nr, rpt):
    hch = ch // 2

    @functools.partial(
        pl.kernel,
        out_type=jax.ShapeDtypeStruct((NC, nr, F), jnp.float32),
        mesh=_MESH,
        scratch_types=[
            pltpu.VMEM((CK,), jnp.int32),
            pltpu.VMEM((CK,), jnp.int32),
            pltpu.VMEM((CK, F), jnp.float32),
            pltpu.VMEM_SHARED((nr, F), jnp.float32),
            pltpu.SemaphoreType.DMA,
            pltpu.SemaphoreType.DMA,
        ],
    )
    def deg_kernel(dst_hbm, zeros_hbm, ones_hbm, out, dst_v, dst_w, ones_v,
                   acc, sem_a, sem_b):
        c = lax.axis_index("c")
        s = lax.axis_index("s")

        pltpu.sync_copy(zeros_hbm, acc.at[pl.ds(s * rpt, rpt)])
        pltpu.sync_copy(ones_hbm, ones_v)
        plsc.subcore_barrier()

        base0 = (c * NS + s) * hch * CK
        pltpu.async_copy(dst_hbm.at[pl.ds(base0, CK)], dst_v, sem_a)
        pltpu.async_copy(dst_hbm.at[pl.ds(base0 + CK, CK)], dst_w, sem_b)

        def body(i, carry):
            j0 = 2 * i
            pltpu.make_async_copy(dst_hbm.at[pl.ds(base0, CK)],
                                  dst_v, sem_a).wait()
            pltpu.sync_copy(ones_v, acc.at[dst_v], add=True)

            @pl.when(i + 1 < hch // 2)
            def _():
                pltpu.async_copy(dst_hbm.at[pl.ds(base0 + (j0 + 2) * CK, CK)],
                                 dst_v, sem_a)

            pltpu.make_async_copy(dst_hbm.at[pl.ds(base0, CK)],
                                  dst_w, sem_b).wait()
            pltpu.sync_copy(ones_v, acc.at[dst_w], add=True)

            @pl.when(i + 1 < hch // 2)
            def _():
                pltpu.async_copy(dst_hbm.at[pl.ds(base0 + (j0 + 3) * CK, CK)],
                                 dst_w, sem_b)
            return carry

        lax.fori_loop(0, hch // 2, body, 0)
        plsc.subcore_barrier()
        pltpu.sync_copy(acc.at[pl.ds(s * rpt, rpt)],
                        out.at[c, pl.ds(s * rpt, rpt)])

    return deg_kernel


def _make_prop_kernel(ch, nr, rpt):
    nhalf = ch // 2

    @functools.partial(
        pl.kernel,
        out_type=jax.ShapeDtypeStruct((NC, nr, F), jnp.float32),
        mesh=_MESH,
        scratch_types=[
            pltpu.VMEM((CK,), jnp.int32),
            pltpu.VMEM((CK,), jnp.int32),
            pltpu.VMEM((CK,), jnp.int32),
            pltpu.VMEM((CK,), jnp.int32),
            pltpu.VMEM((CK, F), jnp.float32),
            pltpu.VMEM((CK, F), jnp.float32),
            pltpu.VMEM_SHARED((nr, F), jnp.float32),
            pltpu.SemaphoreType.DMA,
            pltpu.SemaphoreType.DMA,
            pltpu.SemaphoreType.DMA,
            pltpu.SemaphoreType.DMA,
        ],
    )
    def prop_kernel(src_hbm, dst_hbm, y_a, y_b, zeros_hbm, out,
                    src_a, dst_a, src_b, dst_b, rows_a, rows_b, acc,
                    sem_ia, sem_ib, sem_ga, sem_gb):
        c = lax.axis_index("c")
        s = lax.axis_index("s")

        pltpu.sync_copy(zeros_hbm, acc.at[pl.ds(s * rpt, rpt)])
        plsc.subcore_barrier()

        def run(y_hbm):
            pltpu.async_copy(src_hbm.at[s, 0], src_a, sem_ia)
            pltpu.async_copy(dst_hbm.at[s, 0], dst_a, sem_ia)
            pltpu.async_copy(src_hbm.at[s, 1], src_b, sem_ib)
            pltpu.async_copy(dst_hbm.at[s, 1], dst_b, sem_ib)

            def body(i, carry):
                j0 = 2 * i
                pltpu.make_async_copy(src_hbm.at[s, j0], src_a, sem_ia).wait()
                pltpu.make_async_copy(dst_hbm.at[s, j0], dst_a, sem_ia).wait()
                pltpu.async_copy(y_hbm.at[src_a], rows_a, sem_ga)
                pltpu.make_async_copy(src_hbm.at[s, j0], src_b, sem_ib).wait()
                pltpu.make_async_copy(dst_hbm.at[s, j0], dst_b, sem_ib).wait()
                pltpu.async_copy(y_hbm.at[src_b], rows_b, sem_gb)
                pltpu.make_async_copy(y_hbm.at[src_a], rows_a, sem_ga).wait()
                pltpu.sync_copy(rows_a, acc.at[dst_a], add=True)

                @pl.when(i + 1 < nhalf)
                def _():
                    pltpu.async_copy(src_hbm.at[s, j0 + 2], src_a, sem_ia)
                    pltpu.async_copy(dst_hbm.at[s, j0 + 2], dst_a, sem_ia)

                pltpu.make_async_copy(y_hbm.at[src_b], rows_b, sem_gb).wait()
                pltpu.sync_copy(rows_b, acc.at[dst_b], add=True)

                @pl.when(i + 1 < nhalf)
                def _():
                    pltpu.async_copy(src_hbm.at[s, j0 + 3], src_b, sem_ib)
                    pltpu.async_copy(dst_hbm.at[s, j0 + 3], dst_b, sem_ib)
                return carry

            lax.fori_loop(0, nhalf, body, 0)

        @pl.when(c == 0)
        def _():
            run(y_a)

        @pl.when(c == 1)
        def _():
            run(y_b)

        plsc.subcore_barrier()
        pltpu.sync_copy(acc.at[pl.ds(s * rpt, rpt)],
                        out.at[c, pl.ds(s * rpt, rpt)])

    return prop_kernel


def _mm1(x, w1, deg_a, deg_b):
    n, k = x.shape
    bm = 1000
    grid = n // bm

    def body(x_ref, w_ref, da_ref, db_ref, oa_ref, ob_ref, od_ref):
        xw = jnp.dot(x_ref[...], w_ref[...], preferred_element_type=jnp.float32)
        dinv = lax.rsqrt(da_ref[...] + db_ref[...] + 1.0)
        od_ref[...] = dinv
        y = xw * dinv
        oa_ref[...] = y[:, :F]
        ob_ref[...] = y[:, F:]

    return pl.pallas_call(
        body,
        grid=(grid,),
        in_specs=[
            pl.BlockSpec((bm, k), lambda i: (i, 0)),
            pl.BlockSpec((k, 2 * F), lambda i: (0, 0)),
            pl.BlockSpec((bm, 1), lambda i: (i, 0)),
            pl.BlockSpec((bm, 1), lambda i: (i, 0)),
        ],
        out_specs=[pl.BlockSpec((bm, F), lambda i: (i, 0))] * 2
        + [pl.BlockSpec((bm, 1), lambda i: (i, 0))],
        out_shape=(jax.ShapeDtypeStruct((n, F), jnp.float32),) * 2
        + (jax.ShapeDtypeStruct((n, 1), jnp.float32),),
    )(x, w1, deg_a, deg_b)


def _mm2(s_a, s_b, y_a, y_b, dinv_in, b1, wmu, wlv):
    n = s_a.shape[0]
    bm = 1000
    grid = n // bm

    def body(sa_ref, sb_ref, ya_ref, yb_ref, dinv_ref, b1_ref, wmu_ref,
             wlv_ref, oa_ref, ob_ref):
        dinv = dinv_ref[...]
        b = b1_ref[...]
        ha = jnp.maximum(dinv * (sa_ref[...] + ya_ref[...]) + b[:, :F], 0.0)
        hb = jnp.maximum(dinv * (sb_ref[...] + yb_ref[...]) + b[:, F:], 0.0)
        h = jnp.concatenate([ha, hb], axis=1)
        oa_ref[...] = dinv * jnp.dot(h, wmu_ref[...],
                                     preferred_element_type=jnp.float32)
        ob_ref[...] = dinv * jnp.dot(h, wlv_ref[...],
                                     preferred_element_type=jnp.float32)

    row = lambda i: (i, 0)
    fixed = lambda i: (0, 0)
    return pl.pallas_call(
        body,
        grid=(grid,),
        in_specs=[
            pl.BlockSpec((bm, F), row),
            pl.BlockSpec((bm, F), row),
            pl.BlockSpec((bm, F), row),
            pl.BlockSpec((bm, F), row),
            pl.BlockSpec((bm, 1), row),
            pl.BlockSpec((1, 2 * F), fixed),
            pl.BlockSpec((2 * F, F), fixed),
            pl.BlockSpec((2 * F, F), fixed),
        ],
        out_specs=[pl.BlockSpec((bm, F), row)] * 2,
        out_shape=(jax.ShapeDtypeStruct((n, F), jnp.float32),) * 2,
    )(s_a, s_b, y_a, y_b, dinv_in, b1, wmu, wlv)


def _fin(s_a, s_b, y_a, y_b, dinv_in, bmu, blv):
    n = s_a.shape[0]
    bm = 1000
    grid = n // bm

    def body(sa_ref, sb_ref, ya_ref, yb_ref, dinv_ref, bmu_ref, blv_ref,
             omu_ref, olv_ref):
        dinv = dinv_ref[...]
        omu_ref[...] = dinv * (sa_ref[...] + ya_ref[...]) + bmu_ref[...]
        olv_ref[...] = dinv * (sb_ref[...] + yb_ref[...]) + blv_ref[...]

    row = lambda i: (i, 0)
    fixed = lambda i: (0, 0)
    return pl.pallas_call(
        body,
        grid=(grid,),
        in_specs=[
            pl.BlockSpec((bm, F), row),
            pl.BlockSpec((bm, F), row),
            pl.BlockSpec((bm, F), row),
            pl.BlockSpec((bm, F), row),
            pl.BlockSpec((bm, 1), row),
            pl.BlockSpec((1, F), fixed),
            pl.BlockSpec((1, F), fixed),
        ],
        out_specs=[pl.BlockSpec((bm, F), row)] * 2,
        out_shape=(jax.ShapeDtypeStruct((n, F), jnp.float32),) * 2,
    )(s_a, s_b, y_a, y_b, dinv_in, bmu, blv)


def kernel(x, edge_index, W1, b1, Wmu, bmu, Wlv, blv):
    n = x.shape[0]
    e0 = edge_index.shape[1]

    ch = -(-e0 // (NS * CK))
    ch = (ch + 3) // 4 * 4
    ep = NS * ch * CK
    rpt = ((n + NS - 1) // NS + 15) // 16 * 16
    nr = NS * rpt
    trash = n

    src = edge_index[0].astype(jnp.int32)
    dst = edge_index[1].astype(jnp.int32)
    pad = ep - e0
    pad_iota = jnp.arange(pad, dtype=jnp.int32)
    src_p = jnp.concatenate([src, pad_iota % n])
    dst_p = jnp.concatenate([dst, trash + pad_iota % (nr - n)])
    zeros128 = jnp.zeros((rpt, F), jnp.float32)
    ones128 = jnp.ones((CK, F), jnp.float32)

    deg_kernel = _make_deg_kernel(ch, nr, rpt)
    prop_kernel = _make_prop_kernel(ch, nr, rpt)

    deg_p = deg_kernel(dst_p, zeros128, ones128)
    y1a, y1b, dinv = _mm1(x, W1, deg_p[0, :n, :1], deg_p[1, :n, :1])
    src3 = src_p.reshape(NS, ch, CK)
    dst3 = dst_p.reshape(NS, ch, CK)
    s1 = prop_kernel(src3, dst3, y1a, y1b, zeros128)
    y2a, y2b = _mm2(s1[0, :n], s1[1, :n], y1a, y1b, dinv,
                    b1.reshape(1, 2 * F), Wmu, Wlv)
    s2 = prop_kernel(src3, dst3, y2a, y2b, zeros128)
    mu, logvar = _fin(s2[0, :n], s2[1, :n], y2a, y2b, dinv,
                      bmu.reshape(1, F), blv.reshape(1, F))
    return (mu, logvar)
